# Initial kernel scaffold; baseline (speedup 1.0000x reference)
import jax
import jax.numpy as jnp
from jax.experimental import pallas as pl


def kernel(x, pos, edge_index, W_msg, V_msg, b_msg, W_rbf, b_rbf, W_u1, b_u1, g1, be1, W_u2, b_u2, g2, be2, W_out, b_out):
    raise NotImplementedError("write your pallas kernel here")


# speedup vs baseline: 10.2521x; 10.2521x over previous
import functools

import jax
import jax.numpy as jnp
from jax import lax
from jax.experimental import pallas as pl
from jax.experimental.pallas import tpu as pltpu
from jax.experimental.pallas import tpu_sc as plsc

_N = 10000
_E = 320000
_D = 128
_K = 64
_RMAX = 30.0
_NP = 10112
_RW = 128
_CH = 128
_CPT = 80
_EPAD = 32 * _CPT * _CH
_RPT = _NP // 16

_f32 = jnp.float32
_i32 = jnp.int32

_MESH = plsc.VectorSubcoreMesh(core_axis_name="c", subcore_axis_name="s")


def _zero_fill(zbuf, rows, width):
    zv = jnp.zeros((16,), _f32)
    for i in range(rows):
        for j in range(width // 16):
            zbuf[i, pl.ds(16 * j, 16)] = zv


_IBLK = 8
_RCH = 64
_RCPT = _EPAD // (32 * _RCH)


@functools.partial(
    pl.kernel,
    out_type=jax.ShapeDtypeStruct((2, _NP, _RW), _f32),
    mesh=_MESH,
    scratch_types=[
        pltpu.VMEM((_NP,), _f32),
        pltpu.VMEM((_NP,), _f32),
        pltpu.VMEM((_NP,), _f32),
        pltpu.VMEM((_IBLK, _RCH), _i32),
        pltpu.VMEM((_IBLK, _RCH), _i32),
        pltpu.VMEM((_RCH, _RW), _f32),
        pltpu.VMEM((8, _RW), _f32),
        pltpu.VMEM_SHARED((_NP, _RW), _f32),
    ],
    compiler_params=pltpu.CompilerParams(needs_layout_passes=False),
)
def _sc_rbf_deg(px_hbm, py_hbm, pz_hbm, src_hbm, dst_hbm, out_hbm,
                px, py, pz, srcb, dstb, rows, zbuf, acc):
    c = lax.axis_index("c")
    s = lax.axis_index("s")
    wid = s * 2 + c

    pltpu.sync_copy(px_hbm, px)
    pltpu.sync_copy(py_hbm, py)
    pltpu.sync_copy(pz_hbm, pz)

    _zero_fill(zbuf, 8, _RW)

    def zbody(i, carry):
        pltpu.sync_copy(zbuf, acc.at[pl.ds(s * _RPT + i * 8, 8)])
        return carry

    lax.fori_loop(0, _RPT // 8, zbody, 0)

    lane_i = lax.iota(_i32, 16)
    lane_f = lane_i.astype(_f32)
    centers = [(lane_f + (16.0 * g)) * (_RMAX / (_K - 1)) for g in range(4)]
    inv_w = float(_K) / _RMAX
    zvec = jnp.zeros((16,), _f32)
    deg_one = jnp.where(lane_i == 0, 1.0, 0.0).astype(_f32)

    def const_cols(r, carry):
        rows[r, pl.ds(64, 16)] = deg_one
        rows[r, pl.ds(80, 16)] = zvec
        rows[r, pl.ds(96, 16)] = zvec
        rows[r, pl.ds(112, 16)] = zvec
        return carry

    lax.fori_loop(0, _RCH, const_cols, 0)
    plsc.subcore_barrier()

    def blk_body(bi, carry):
        base = pl.multiple_of(wid * _RCPT + bi * _IBLK, _IBLK)
        pltpu.sync_copy(src_hbm.at[pl.ds(base, _IBLK)], srcb)
        pltpu.sync_copy(dst_hbm.at[pl.ds(base, _IBLK)], dstb)

        def chunk_body(cj, carry2):
            def group_body(g, gcarry):
                off = pl.multiple_of(16 * g, 16)
                si = srcb[cj, pl.ds(off, 16)]
                di = dstb[cj, pl.ds(off, 16)]
                dx = plsc.load_gather(px, [di]) - plsc.load_gather(px, [si])
                dy = plsc.load_gather(py, [di]) - plsc.load_gather(py, [si])
                dz = plsc.load_gather(pz, [di]) - plsc.load_gather(pz, [si])
                dd = dx * dx + dy * dy + dz * dz + 1e-12
                bits = lax.bitcast_convert_type(dd, _i32)
                y = lax.bitcast_convert_type(
                    lax.shift_right_logical(bits, 1) + 0x1FBD1DF6, _f32)
                for _ in range(3):
                    y = 0.5 * (y + dd / y)
                for el in range(16):
                    dv = y.at[jnp.full((16,), el, _i32)].get(
                        mode="promise_in_bounds")
                    for j in range(4):
                        t = (dv - centers[j]) * inv_w
                        rows[off + el, pl.ds(16 * j, 16)] = (
                            jnp.exp(-0.5 * t * t))
                return gcarry

            lax.fori_loop(0, _RCH // 16, group_body, 0)
            pltpu.sync_copy(rows, acc.at[dstb.at[cj]], add=True)
            return carry2

        lax.fori_loop(0, _IBLK, chunk_body, 0)
        return carry

    lax.fori_loop(0, _RCPT // _IBLK, blk_body, 0)
    plsc.subcore_barrier()
    pltpu.sync_copy(acc.at[pl.ds(s * _RPT, _RPT)],
                    out_hbm.at[c, pl.ds(s * _RPT, _RPT)])


@functools.partial(
    pl.kernel,
    out_type=jax.ShapeDtypeStruct((2, _NP, _D), _f32),
    mesh=_MESH,
    scratch_types=[
        pltpu.VMEM((_CPT, _CH), _i32),
        pltpu.VMEM((_CPT, _CH), _i32),
        pltpu.VMEM((_CH, _D), _f32),
        pltpu.VMEM((8, _D), _f32),
        pltpu.VMEM_SHARED((_NP, _D), _f32),
        pltpu.SemaphoreType.DMA,
    ],
    compiler_params=pltpu.CompilerParams(needs_layout_passes=False),
)
def _sc_scatter_rows(tbl_hbm, src_hbm, dst_hbm, out_hbm,
                     srcb, dstb, rowsv, zbuf, acc, sem):
    c = lax.axis_index("c")
    s = lax.axis_index("s")
    wid = s * 2 + c

    pltpu.sync_copy(src_hbm.at[pl.ds(wid * _CPT, _CPT)], srcb)
    pltpu.sync_copy(dst_hbm.at[pl.ds(wid * _CPT, _CPT)], dstb)

    _zero_fill(zbuf, 8, _D)

    def zbody(i, carry):
        pltpu.sync_copy(zbuf, acc.at[pl.ds(s * _RPT + i * 8, 8)])
        return carry

    lax.fori_loop(0, _RPT // 8, zbody, 0)
    plsc.subcore_barrier()

    def chunk_body(ci, carry):
        pltpu.async_copy(tbl_hbm.at[srcb.at[ci]], rowsv, sem).wait()
        pltpu.sync_copy(rowsv, acc.at[dstb.at[ci]], add=True)
        return carry

    lax.fori_loop(0, _CPT, chunk_body, 0)
    plsc.subcore_barrier()
    pltpu.sync_copy(acc.at[pl.ds(s * _RPT, _RPT)],
                    out_hbm.at[c, pl.ds(s * _RPT, _RPT)])


_BLK = 1264


def _mm_body(x_ref, w_ref, o_ref):
    o_ref[...] = jnp.dot(x_ref[...], w_ref[...],
                         preferred_element_type=_f32)


def _tc_mm(x, w):
    return pl.pallas_call(
        _mm_body,
        grid=(_NP // _BLK,),
        in_specs=[
            pl.BlockSpec((_BLK, _D), lambda i: (i, 0)),
            pl.BlockSpec((_D, _D), lambda i: (0, 0)),
        ],
        out_specs=pl.BlockSpec((_BLK, _D), lambda i: (i, 0)),
        out_shape=jax.ShapeDtypeStruct((_NP, _D), _f32),
    )(x, w)


def _ln_relu(u, g, b):
    mu = jnp.mean(u, axis=-1, keepdims=True)
    d = u - mu
    var = jnp.mean(d * d, axis=-1, keepdims=True)
    z = d / jnp.sqrt(var + 1e-5) * g + b
    return jnp.maximum(z, 0.0)


def _upd_body(h_ref, sp_ref, rp_ref, wr_ref, v_ref, brbf_ref, bmsg_ref,
              wd_ref, w1h_ref, w1a_ref, b1_ref, g1_ref, be1_ref,
              w2_ref, b2_ref, g2_ref, be2_ref, wn_ref, bn_ref,
              hn_ref, yn_ref):
    dot = functools.partial(jnp.dot, preferred_element_type=_f32)
    sagg = sp_ref[0] + sp_ref[1]
    rq = rp_ref[0] + rp_ref[1]
    rm = rq[:, :_K]
    deg = rq[:, _K:_K + 1]
    wrv = dot(wr_ref[...], v_ref[...])
    cvec = dot(brbf_ref[...], v_ref[...]) + bmsg_ref[...]
    h = h_ref[...]
    a = dot(h, wd_ref[...])
    aggr = sagg + deg * a + deg * cvec + dot(rm, wrv)
    u = dot(h, w1h_ref[...]) + dot(aggr, w1a_ref[...]) + b1_ref[...]
    z = _ln_relu(u, g1_ref[...], be1_ref[...])
    z = _ln_relu(dot(z, w2_ref[...]) + b2_ref[...], g2_ref[...], be2_ref[...])
    hn = jnp.where(deg > 0.0, h + z, h)
    hn_ref[...] = hn
    yn_ref[...] = dot(hn, wn_ref[...]) + bn_ref[...]


def _tc_update(h, s_parts, r_parts, w_rbf, v_msg, b_rbf, b_msg,
               wd, w1h, w1a, b1, g1, be1, w2, b2, g2, be2, wn, bn):
    row = lambda i: (i, 0)
    part = lambda i: (0, i, 0)
    fixed2 = lambda i: (0, 0)
    return pl.pallas_call(
        _upd_body,
        grid=(_NP // _BLK,),
        in_specs=[
            pl.BlockSpec((_BLK, _D), row),
            pl.BlockSpec((2, _BLK, _D), part),
            pl.BlockSpec((2, _BLK, _RW), part),
            pl.BlockSpec((_K, _D), fixed2),
            pl.BlockSpec((_D, _D), fixed2),
            pl.BlockSpec((1, _D), fixed2),
            pl.BlockSpec((1, _D), fixed2),
            pl.BlockSpec((_D, _D), fixed2),
            pl.BlockSpec((_D, _D), fixed2),
            pl.BlockSpec((_D, _D), fixed2),
            pl.BlockSpec((1, _D), fixed2),
            pl.BlockSpec((1, _D), fixed2),
            pl.BlockSpec((1, _D), fixed2),
            pl.BlockSpec((_D, _D), fixed2),
            pl.BlockSpec((1, _D), fixed2),
            pl.BlockSpec((1, _D), fixed2),
            pl.BlockSpec((1, _D), fixed2),
            pl.BlockSpec((_D, _D), fixed2),
            pl.BlockSpec((1, _D), fixed2),
        ],
        out_specs=[
            pl.BlockSpec((_BLK, _D), row),
            pl.BlockSpec((_BLK, _D), row),
        ],
        out_shape=[
            jax.ShapeDtypeStruct((_NP, _D), _f32),
            jax.ShapeDtypeStruct((_NP, _D), _f32),
        ],
    )(h, s_parts, r_parts, w_rbf, v_msg, b_rbf, b_msg,
      wd, w1h, w1a, b1, g1, be1, w2, b2, g2, be2, wn, bn)


def kernel(x, pos, edge_index, W_msg, V_msg, b_msg, W_rbf, b_rbf,
           W_u1, b_u1, g1, be1, W_u2, b_u2, g2, be2, W_out, b_out):
    src = edge_index[0].astype(_i32)
    dst = edge_index[1].astype(_i32)
    pad = _N + (jnp.arange(_EPAD - _E, dtype=_i32) % (_NP - _N))
    src2 = jnp.concatenate([src, pad]).reshape(_EPAD // _CH, _CH)
    dst2 = jnp.concatenate([dst, pad]).reshape(_EPAD // _CH, _CH)
    src2r = src2.reshape(_EPAD // _RCH, _RCH)
    dst2r = dst2.reshape(_EPAD // _RCH, _RCH)
    zpad = jnp.zeros((_NP - _N,), _f32)
    px = jnp.concatenate([pos[:, 0], zpad])
    py = jnp.concatenate([pos[:, 1], zpad])
    pz = jnp.concatenate([pos[:, 2], zpad])
    xp = jnp.zeros((_NP, _D), _f32).at[:_N].set(x)

    r_parts = _sc_rbf_deg(px, py, pz, src2r, dst2r)

    def row2(b):
        return b.reshape(1, _D)

    b0 = _tc_mm(xp, W_msg[0, _D:])
    s_parts = _sc_scatter_rows(b0, src2, dst2)
    h1, b1t = _tc_update(
        xp, s_parts, r_parts, W_rbf[0], V_msg[0], row2(b_rbf[0]),
        row2(b_msg[0]), W_msg[0, :_D], W_u1[0, :_D], W_u1[0, _D:],
        row2(b_u1[0]), row2(g1[0]), row2(be1[0]), W_u2[0], row2(b_u2[0]),
        row2(g2[0]), row2(be2[0]), W_msg[1, _D:], jnp.zeros((1, _D), _f32))
    s_parts1 = _sc_scatter_rows(b1t, src2, dst2)
    _, out = _tc_update(
        h1, s_parts1, r_parts, W_rbf[1], V_msg[1], row2(b_rbf[1]),
        row2(b_msg[1]), W_msg[1, :_D], W_u1[1, :_D], W_u1[1, _D:],
        row2(b_u1[1]), row2(g1[1]), row2(be1[1]), W_u2[1], row2(b_u2[1]),
        row2(g2[1]), row2(be2[1]), W_out, row2(b_out))
    return out[:_N]


# speedup vs baseline: 11.9909x; 1.1696x over previous
import functools

import jax
import jax.numpy as jnp
from jax import lax
from jax.experimental import pallas as pl
from jax.experimental.pallas import tpu as pltpu
from jax.experimental.pallas import tpu_sc as plsc

_N = 10000
_E = 320000
_D = 128
_K = 64
_RMAX = 30.0
_NP = 10112
_RW = 128
_CH = 128
_CPT = 80
_EPAD = 32 * _CPT * _CH
_RPT = _NP // 16

_f32 = jnp.float32
_i32 = jnp.int32

_MESH = plsc.VectorSubcoreMesh(core_axis_name="c", subcore_axis_name="s")


def _zero_fill(zbuf, rows, width):
    zv = jnp.zeros((16,), _f32)
    for i in range(rows):
        for j in range(width // 16):
            zbuf[i, pl.ds(16 * j, 16)] = zv


_IBLK = 8
_RCH = 64
_RCPT = _EPAD // (32 * _RCH)


@functools.partial(
    pl.kernel,
    out_type=jax.ShapeDtypeStruct((2, _NP, _RW), _f32),
    mesh=_MESH,
    scratch_types=[
        pltpu.VMEM((_NP,), _f32),
        pltpu.VMEM((_NP,), _f32),
        pltpu.VMEM((_NP,), _f32),
        pltpu.VMEM((_IBLK, _RCH), _i32),
        pltpu.VMEM((_IBLK, _RCH), _i32),
        pltpu.VMEM((_RCH, _RW), _f32),
        pltpu.VMEM((8, _RW), _f32),
        pltpu.VMEM_SHARED((_NP, _RW), _f32),
    ],
    compiler_params=pltpu.CompilerParams(needs_layout_passes=False),
)
def _sc_rbf_deg(px_hbm, py_hbm, pz_hbm, src_hbm, dst_hbm, out_hbm,
                px, py, pz, srcb, dstb, rows, zbuf, acc):
    c = lax.axis_index("c")
    s = lax.axis_index("s")
    wid = s * 2 + c

    pltpu.sync_copy(px_hbm, px)
    pltpu.sync_copy(py_hbm, py)
    pltpu.sync_copy(pz_hbm, pz)

    _zero_fill(zbuf, 8, _RW)

    def zbody(i, carry):
        pltpu.sync_copy(zbuf, acc.at[pl.ds(s * _RPT + i * 8, 8)])
        return carry

    lax.fori_loop(0, _RPT // 8, zbody, 0)

    lane_i = lax.iota(_i32, 16)
    lane_f = lane_i.astype(_f32)
    centers = [(lane_f + (16.0 * g)) * (_RMAX / (_K - 1)) for g in range(4)]
    inv_w = float(_K) / _RMAX
    zvec = jnp.zeros((16,), _f32)
    deg_one = jnp.where(lane_i == 0, 1.0, 0.0).astype(_f32)

    def const_cols(r, carry):
        rows[r, pl.ds(64, 16)] = deg_one
        rows[r, pl.ds(80, 16)] = zvec
        rows[r, pl.ds(96, 16)] = zvec
        rows[r, pl.ds(112, 16)] = zvec
        return carry

    lax.fori_loop(0, _RCH, const_cols, 0)
    plsc.subcore_barrier()

    def blk_body(bi, carry):
        base = pl.multiple_of(wid * _RCPT + bi * _IBLK, _IBLK)
        pltpu.sync_copy(src_hbm.at[pl.ds(base, _IBLK)], srcb)
        pltpu.sync_copy(dst_hbm.at[pl.ds(base, _IBLK)], dstb)

        def chunk_body(cj, carry2):
            def group_body(g, gcarry):
                off = pl.multiple_of(16 * g, 16)
                si = srcb[cj, pl.ds(off, 16)]
                di = dstb[cj, pl.ds(off, 16)]
                dx = plsc.load_gather(px, [di]) - plsc.load_gather(px, [si])
                dy = plsc.load_gather(py, [di]) - plsc.load_gather(py, [si])
                dz = plsc.load_gather(pz, [di]) - plsc.load_gather(pz, [si])
                dd = dx * dx + dy * dy + dz * dz + 1e-12
                bits = lax.bitcast_convert_type(dd, _i32)
                y = lax.bitcast_convert_type(
                    lax.shift_right_logical(bits, 1) + 0x1FBD1DF6, _f32)
                for _ in range(3):
                    y = 0.5 * (y + dd / y)
                for el in range(16):
                    dv = y.at[jnp.full((16,), el, _i32)].get(
                        mode="promise_in_bounds")
                    for j in range(4):
                        t = (dv - centers[j]) * inv_w
                        rows[off + el, pl.ds(16 * j, 16)] = (
                            jnp.exp(-0.5 * t * t))
                return gcarry

            lax.fori_loop(0, _RCH // 16, group_body, 0)
            pltpu.sync_copy(rows, acc.at[dstb.at[cj]], add=True)
            return carry2

        lax.fori_loop(0, _IBLK, chunk_body, 0)
        return carry

    lax.fori_loop(0, _RCPT // _IBLK, blk_body, 0)
    plsc.subcore_barrier()
    pltpu.sync_copy(acc.at[pl.ds(s * _RPT, _RPT)],
                    out_hbm.at[c, pl.ds(s * _RPT, _RPT)])


_SBLK = 8


@functools.partial(
    pl.kernel,
    out_type=jax.ShapeDtypeStruct((2, _NP, _D), _f32),
    mesh=_MESH,
    scratch_types=[
        pltpu.VMEM((_SBLK, _CH), _i32),
        pltpu.VMEM((_SBLK, _CH), _i32),
        pltpu.VMEM((_CH, _D), _f32),
        pltpu.VMEM((_CH, _D), _f32),
        pltpu.VMEM((8, _D), _f32),
        pltpu.VMEM_SHARED((_NP, _D), _f32),
        pltpu.SemaphoreType.DMA,
        pltpu.SemaphoreType.DMA,
    ],
    compiler_params=pltpu.CompilerParams(needs_layout_passes=False),
)
def _sc_scatter_rows(tbl_hbm, src_hbm, dst_hbm, out_hbm,
                     srcb, dstb, rows0, rows1, zbuf, acc, sem0, sem1):
    c = lax.axis_index("c")
    s = lax.axis_index("s")
    wid = s * 2 + c

    _zero_fill(zbuf, 8, _D)

    def zbody(i, carry):
        pltpu.sync_copy(zbuf, acc.at[pl.ds(s * _RPT + i * 8, 8)])
        return carry

    lax.fori_loop(0, _RPT // 8, zbody, 0)
    plsc.subcore_barrier()

    bufs = (rows0, rows1)
    sems = (sem0, sem1)

    def blk_body(bi, carry):
        base = pl.multiple_of(wid * _CPT + bi * _SBLK, _SBLK)
        pltpu.sync_copy(src_hbm.at[pl.ds(base, _SBLK)], srcb)
        pltpu.sync_copy(dst_hbm.at[pl.ds(base, _SBLK)], dstb)
        pend = pltpu.async_copy(tbl_hbm.at[srcb.at[0]], bufs[0], sems[0])
        for cj in range(_SBLK):
            nxt = None
            if cj + 1 < _SBLK:
                nxt = pltpu.async_copy(
                    tbl_hbm.at[srcb.at[cj + 1]],
                    bufs[(cj + 1) % 2], sems[(cj + 1) % 2])
            pend.wait()
            pltpu.sync_copy(bufs[cj % 2], acc.at[dstb.at[cj]], add=True)
            pend = nxt
        return carry

    lax.fori_loop(0, _CPT // _SBLK, blk_body, 0)
    plsc.subcore_barrier()
    pltpu.sync_copy(acc.at[pl.ds(s * _RPT, _RPT)],
                    out_hbm.at[c, pl.ds(s * _RPT, _RPT)])


_BLK = 1264


def _mm_body(x_ref, w_ref, o_ref):
    o_ref[...] = jnp.dot(x_ref[...], w_ref[...],
                         preferred_element_type=_f32)


def _tc_mm(x, w):
    return pl.pallas_call(
        _mm_body,
        grid=(_NP // _BLK,),
        in_specs=[
            pl.BlockSpec((_BLK, _D), lambda i: (i, 0)),
            pl.BlockSpec((_D, _D), lambda i: (0, 0)),
        ],
        out_specs=pl.BlockSpec((_BLK, _D), lambda i: (i, 0)),
        out_shape=jax.ShapeDtypeStruct((_NP, _D), _f32),
    )(x, w)


def _ln_relu(u, g, b):
    mu = jnp.mean(u, axis=-1, keepdims=True)
    d = u - mu
    var = jnp.mean(d * d, axis=-1, keepdims=True)
    z = d / jnp.sqrt(var + 1e-5) * g + b
    return jnp.maximum(z, 0.0)


def _upd_body(h_ref, sp_ref, rp_ref, wr_ref, v_ref, brbf_ref, bmsg_ref,
              wd_ref, w1h_ref, w1a_ref, b1_ref, g1_ref, be1_ref,
              w2_ref, b2_ref, g2_ref, be2_ref, wn_ref, bn_ref,
              hn_ref, yn_ref):
    dot = functools.partial(jnp.dot, preferred_element_type=_f32)
    sagg = sp_ref[0] + sp_ref[1]
    rq = rp_ref[0] + rp_ref[1]
    rm = rq[:, :_K]
    deg = rq[:, _K:_K + 1]
    wrv = dot(wr_ref[...], v_ref[...])
    cvec = dot(brbf_ref[...], v_ref[...]) + bmsg_ref[...]
    h = h_ref[...]
    a = dot(h, wd_ref[...])
    aggr = sagg + deg * a + deg * cvec + dot(rm, wrv)
    u = dot(h, w1h_ref[...]) + dot(aggr, w1a_ref[...]) + b1_ref[...]
    z = _ln_relu(u, g1_ref[...], be1_ref[...])
    z = _ln_relu(dot(z, w2_ref[...]) + b2_ref[...], g2_ref[...], be2_ref[...])
    hn = jnp.where(deg > 0.0, h + z, h)
    hn_ref[...] = hn
    yn_ref[...] = dot(hn, wn_ref[...]) + bn_ref[...]


def _tc_update(h, s_parts, r_parts, w_rbf, v_msg, b_rbf, b_msg,
               wd, w1h, w1a, b1, g1, be1, w2, b2, g2, be2, wn, bn):
    row = lambda i: (i, 0)
    part = lambda i: (0, i, 0)
    fixed2 = lambda i: (0, 0)
    return pl.pallas_call(
        _upd_body,
        grid=(_NP // _BLK,),
        in_specs=[
            pl.BlockSpec((_BLK, _D), row),
            pl.BlockSpec((2, _BLK, _D), part),
            pl.BlockSpec((2, _BLK, _RW), part),
            pl.BlockSpec((_K, _D), fixed2),
            pl.BlockSpec((_D, _D), fixed2),
            pl.BlockSpec((1, _D), fixed2),
            pl.BlockSpec((1, _D), fixed2),
            pl.BlockSpec((_D, _D), fixed2),
            pl.BlockSpec((_D, _D), fixed2),
            pl.BlockSpec((_D, _D), fixed2),
            pl.BlockSpec((1, _D), fixed2),
            pl.BlockSpec((1, _D), fixed2),
            pl.BlockSpec((1, _D), fixed2),
            pl.BlockSpec((_D, _D), fixed2),
            pl.BlockSpec((1, _D), fixed2),
            pl.BlockSpec((1, _D), fixed2),
            pl.BlockSpec((1, _D), fixed2),
            pl.BlockSpec((_D, _D), fixed2),
            pl.BlockSpec((1, _D), fixed2),
        ],
        out_specs=[
            pl.BlockSpec((_BLK, _D), row),
            pl.BlockSpec((_BLK, _D), row),
        ],
        out_shape=[
            jax.ShapeDtypeStruct((_NP, _D), _f32),
            jax.ShapeDtypeStruct((_NP, _D), _f32),
        ],
    )(h, s_parts, r_parts, w_rbf, v_msg, b_rbf, b_msg,
      wd, w1h, w1a, b1, g1, be1, w2, b2, g2, be2, wn, bn)


def kernel(x, pos, edge_index, W_msg, V_msg, b_msg, W_rbf, b_rbf,
           W_u1, b_u1, g1, be1, W_u2, b_u2, g2, be2, W_out, b_out):
    src = edge_index[0].astype(_i32)
    dst = edge_index[1].astype(_i32)
    pad = _N + (jnp.arange(_EPAD - _E, dtype=_i32) % (_NP - _N))
    src2 = jnp.concatenate([src, pad]).reshape(_EPAD // _CH, _CH)
    dst2 = jnp.concatenate([dst, pad]).reshape(_EPAD // _CH, _CH)
    src2r = src2.reshape(_EPAD // _RCH, _RCH)
    dst2r = dst2.reshape(_EPAD // _RCH, _RCH)
    zpad = jnp.zeros((_NP - _N,), _f32)
    px = jnp.concatenate([pos[:, 0], zpad])
    py = jnp.concatenate([pos[:, 1], zpad])
    pz = jnp.concatenate([pos[:, 2], zpad])
    xp = jnp.zeros((_NP, _D), _f32).at[:_N].set(x)

    r_parts = _sc_rbf_deg(px, py, pz, src2r, dst2r)

    def row2(b):
        return b.reshape(1, _D)

    b0 = _tc_mm(xp, W_msg[0, _D:])
    s_parts = _sc_scatter_rows(b0, src2, dst2)
    h1, b1t = _tc_update(
        xp, s_parts, r_parts, W_rbf[0], V_msg[0], row2(b_rbf[0]),
        row2(b_msg[0]), W_msg[0, :_D], W_u1[0, :_D], W_u1[0, _D:],
        row2(b_u1[0]), row2(g1[0]), row2(be1[0]), W_u2[0], row2(b_u2[0]),
        row2(g2[0]), row2(be2[0]), W_msg[1, _D:], jnp.zeros((1, _D), _f32))
    s_parts1 = _sc_scatter_rows(b1t, src2, dst2)
    _, out = _tc_update(
        h1, s_parts1, r_parts, W_rbf[1], V_msg[1], row2(b_rbf[1]),
        row2(b_msg[1]), W_msg[1, :_D], W_u1[1, :_D], W_u1[1, _D:],
        row2(b_u1[1]), row2(g1[1]), row2(be1[1]), W_u2[1], row2(b_u2[1]),
        row2(g2[1]), row2(be2[1]), W_out, row2(b_out))
    return out[:_N]


# speedup vs baseline: 13.1825x; 1.0994x over previous
import functools

import jax
import jax.numpy as jnp
from jax import lax
from jax.experimental import pallas as pl
from jax.experimental.pallas import tpu as pltpu
from jax.experimental.pallas import tpu_sc as plsc

_N = 10000
_E = 320000
_D = 128
_K = 64
_RMAX = 30.0
_NP = 10112
_RW = 128
_CH = 128
_CPT = 80
_EPAD = 32 * _CPT * _CH
_RPT = _NP // 16

_f32 = jnp.float32
_i32 = jnp.int32

_MESH = plsc.VectorSubcoreMesh(core_axis_name="c", subcore_axis_name="s")


def _zero_fill(zbuf, rows, width):
    zv = jnp.zeros((16,), _f32)
    for i in range(rows):
        for j in range(width // 16):
            zbuf[i, pl.ds(16 * j, 16)] = zv


_IBLK = 8
_RCH = 64
_RCPT = _EPAD // (32 * _RCH)


@functools.partial(
    pl.kernel,
    out_type=jax.ShapeDtypeStruct((2, _NP, _RW), _f32),
    mesh=_MESH,
    scratch_types=[
        pltpu.VMEM((_NP,), _f32),
        pltpu.VMEM((_NP,), _f32),
        pltpu.VMEM((_NP,), _f32),
        pltpu.VMEM((_IBLK, _RCH), _i32),
        pltpu.VMEM((_IBLK, _RCH), _i32),
        pltpu.VMEM((_RCH, _RW), _f32),
        pltpu.VMEM((_RCH, _RW), _f32),
        pltpu.VMEM((8, _RW), _f32),
        pltpu.VMEM_SHARED((_NP, _RW), _f32),
        pltpu.SemaphoreType.DMA,
        pltpu.SemaphoreType.DMA,
    ],
    compiler_params=pltpu.CompilerParams(needs_layout_passes=False),
)
def _sc_rbf_deg(px_hbm, py_hbm, pz_hbm, src_hbm, dst_hbm, out_hbm,
                px, py, pz, srcb, dstb, rows0, rows1, zbuf, acc,
                sem0, sem1):
    c = lax.axis_index("c")
    s = lax.axis_index("s")
    wid = s * 2 + c

    pltpu.sync_copy(px_hbm, px)
    pltpu.sync_copy(py_hbm, py)
    pltpu.sync_copy(pz_hbm, pz)

    _zero_fill(zbuf, 8, _RW)

    def zbody(i, carry):
        pltpu.sync_copy(zbuf, acc.at[pl.ds(s * _RPT + i * 8, 8)])
        return carry

    lax.fori_loop(0, _RPT // 8, zbody, 0)

    lane_i = lax.iota(_i32, 16)
    lane_f = lane_i.astype(_f32)
    inv_w = float(_K) / _RMAX
    centers_s = [(lane_f + (16.0 * g)) * (_RMAX / (_K - 1) * inv_w)
                 for g in range(4)]
    neg_half = -0.5
    zvec = jnp.zeros((16,), _f32)
    deg_one = jnp.where(lane_i == 0, 1.0, 0.0).astype(_f32)

    bufs = (rows0, rows1)
    sems = (sem0, sem1)

    for rbuf in bufs:
        def const_cols(r, carry, rbuf=rbuf):
            rbuf[r, pl.ds(64, 16)] = deg_one
            rbuf[r, pl.ds(80, 16)] = zvec
            rbuf[r, pl.ds(96, 16)] = zvec
            rbuf[r, pl.ds(112, 16)] = zvec
            return carry

        lax.fori_loop(0, _RCH, const_cols, 0)
    plsc.subcore_barrier()

    def blk_body(bi, carry):
        base = pl.multiple_of(wid * _RCPT + bi * _IBLK, _IBLK)
        pltpu.sync_copy(src_hbm.at[pl.ds(base, _IBLK)], srcb)
        pltpu.sync_copy(dst_hbm.at[pl.ds(base, _IBLK)], dstb)

        for cj in range(_IBLK):
            b = cj % 2
            rows = bufs[b]
            if cj >= 2:
                pltpu.make_async_copy(
                    rows, acc.at[dstb.at[cj]], sems[b]).wait()

            def group_body(g, gcarry, rows=rows, cj=cj):
                off = pl.multiple_of(16 * g, 16)
                si = srcb[cj, pl.ds(off, 16)]
                di = dstb[cj, pl.ds(off, 16)]
                dx = plsc.load_gather(px, [di]) - plsc.load_gather(px, [si])
                dy = plsc.load_gather(py, [di]) - plsc.load_gather(py, [si])
                dz = plsc.load_gather(pz, [di]) - plsc.load_gather(pz, [si])
                dd = dx * dx + dy * dy + dz * dz + 1e-12
                bits = lax.bitcast_convert_type(dd, _i32)
                y = lax.bitcast_convert_type(
                    lax.shift_right_logical(bits, 1) + 0x1FBD1DF6, _f32)
                for _ in range(3):
                    y = 0.5 * (y + dd / y)
                ys = y * inv_w
                for el in range(16):
                    dv = ys.at[jnp.full((16,), el, _i32)].get(
                        mode="promise_in_bounds")
                    for j in range(4):
                        t = dv - centers_s[j]
                        rows[off + el, pl.ds(16 * j, 16)] = (
                            jnp.exp((t * t) * neg_half))
                return gcarry

            lax.fori_loop(0, _RCH // 16, group_body, 0)
            pltpu.async_copy(rows, acc.at[dstb.at[cj]], sems[b], add=True)

        pltpu.make_async_copy(bufs[0], acc.at[dstb.at[_IBLK - 2]],
                              sems[0]).wait()
        pltpu.make_async_copy(bufs[1], acc.at[dstb.at[_IBLK - 1]],
                              sems[1]).wait()
        return carry

    lax.fori_loop(0, _RCPT // _IBLK, blk_body, 0)
    plsc.subcore_barrier()
    pltpu.sync_copy(acc.at[pl.ds(s * _RPT, _RPT)],
                    out_hbm.at[c, pl.ds(s * _RPT, _RPT)])


_SBLK = 8


@functools.partial(
    pl.kernel,
    out_type=jax.ShapeDtypeStruct((2, _NP, _D), _f32),
    mesh=_MESH,
    scratch_types=[
        pltpu.VMEM((_SBLK, _CH), _i32),
        pltpu.VMEM((_SBLK, _CH), _i32),
        pltpu.VMEM((_CH, _D), _f32),
        pltpu.VMEM((_CH, _D), _f32),
        pltpu.VMEM((8, _D), _f32),
        pltpu.VMEM_SHARED((_NP, _D), _f32),
        pltpu.SemaphoreType.DMA,
        pltpu.SemaphoreType.DMA,
    ],
    compiler_params=pltpu.CompilerParams(needs_layout_passes=False),
)
def _sc_scatter_rows(tbl_hbm, src_hbm, dst_hbm, out_hbm,
                     srcb, dstb, rows0, rows1, zbuf, acc, sem0, sem1):
    c = lax.axis_index("c")
    s = lax.axis_index("s")
    wid = s * 2 + c

    _zero_fill(zbuf, 8, _D)

    def zbody(i, carry):
        pltpu.sync_copy(zbuf, acc.at[pl.ds(s * _RPT + i * 8, 8)])
        return carry

    lax.fori_loop(0, _RPT // 8, zbody, 0)
    plsc.subcore_barrier()

    bufs = (rows0, rows1)
    sems = (sem0, sem1)

    def blk_body(bi, carry):
        base = pl.multiple_of(wid * _CPT + bi * _SBLK, _SBLK)
        pltpu.sync_copy(src_hbm.at[pl.ds(base, _SBLK)], srcb)
        pltpu.sync_copy(dst_hbm.at[pl.ds(base, _SBLK)], dstb)
        pend = pltpu.async_copy(tbl_hbm.at[srcb.at[0]], bufs[0], sems[0])
        for cj in range(_SBLK):
            nxt = None
            if cj + 1 < _SBLK:
                nxt = pltpu.async_copy(
                    tbl_hbm.at[srcb.at[cj + 1]],
                    bufs[(cj + 1) % 2], sems[(cj + 1) % 2])
            pend.wait()
            pltpu.sync_copy(bufs[cj % 2], acc.at[dstb.at[cj]], add=True)
            pend = nxt
        return carry

    lax.fori_loop(0, _CPT // _SBLK, blk_body, 0)
    plsc.subcore_barrier()
    pltpu.sync_copy(acc.at[pl.ds(s * _RPT, _RPT)],
                    out_hbm.at[c, pl.ds(s * _RPT, _RPT)])


_BLK = 1264


def _mm_body(x_ref, w_ref, o_ref):
    o_ref[...] = jnp.dot(x_ref[...], w_ref[...],
                         preferred_element_type=_f32)


def _tc_mm(x, w):
    return pl.pallas_call(
        _mm_body,
        grid=(_NP // _BLK,),
        in_specs=[
            pl.BlockSpec((_BLK, _D), lambda i: (i, 0)),
            pl.BlockSpec((_D, _D), lambda i: (0, 0)),
        ],
        out_specs=pl.BlockSpec((_BLK, _D), lambda i: (i, 0)),
        out_shape=jax.ShapeDtypeStruct((_NP, _D), _f32),
    )(x, w)


def _ln_relu(u, g, b):
    mu = jnp.mean(u, axis=-1, keepdims=True)
    d = u - mu
    var = jnp.mean(d * d, axis=-1, keepdims=True)
    z = d / jnp.sqrt(var + 1e-5) * g + b
    return jnp.maximum(z, 0.0)


def _upd_body(h_ref, sp_ref, rp_ref, wr_ref, v_ref, brbf_ref, bmsg_ref,
              wd_ref, w1h_ref, w1a_ref, b1_ref, g1_ref, be1_ref,
              w2_ref, b2_ref, g2_ref, be2_ref, wn_ref, bn_ref,
              hn_ref, yn_ref):
    dot = functools.partial(jnp.dot, preferred_element_type=_f32)
    sagg = sp_ref[0] + sp_ref[1]
    rq = rp_ref[0] + rp_ref[1]
    rm = rq[:, :_K]
    deg = rq[:, _K:_K + 1]
    wrv = dot(wr_ref[...], v_ref[...])
    cvec = dot(brbf_ref[...], v_ref[...]) + bmsg_ref[...]
    h = h_ref[...]
    a = dot(h, wd_ref[...])
    aggr = sagg + deg * a + deg * cvec + dot(rm, wrv)
    u = dot(h, w1h_ref[...]) + dot(aggr, w1a_ref[...]) + b1_ref[...]
    z = _ln_relu(u, g1_ref[...], be1_ref[...])
    z = _ln_relu(dot(z, w2_ref[...]) + b2_ref[...], g2_ref[...], be2_ref[...])
    hn = jnp.where(deg > 0.0, h + z, h)
    hn_ref[...] = hn
    yn_ref[...] = dot(hn, wn_ref[...]) + bn_ref[...]


def _tc_update(h, s_parts, r_parts, w_rbf, v_msg, b_rbf, b_msg,
               wd, w1h, w1a, b1, g1, be1, w2, b2, g2, be2, wn, bn):
    row = lambda i: (i, 0)
    part = lambda i: (0, i, 0)
    fixed2 = lambda i: (0, 0)
    return pl.pallas_call(
        _upd_body,
        grid=(_NP // _BLK,),
        in_specs=[
            pl.BlockSpec((_BLK, _D), row),
            pl.BlockSpec((2, _BLK, _D), part),
            pl.BlockSpec((2, _BLK, _RW), part),
            pl.BlockSpec((_K, _D), fixed2),
            pl.BlockSpec((_D, _D), fixed2),
            pl.BlockSpec((1, _D), fixed2),
            pl.BlockSpec((1, _D), fixed2),
            pl.BlockSpec((_D, _D), fixed2),
            pl.BlockSpec((_D, _D), fixed2),
            pl.BlockSpec((_D, _D), fixed2),
            pl.BlockSpec((1, _D), fixed2),
            pl.BlockSpec((1, _D), fixed2),
            pl.BlockSpec((1, _D), fixed2),
            pl.BlockSpec((_D, _D), fixed2),
            pl.BlockSpec((1, _D), fixed2),
            pl.BlockSpec((1, _D), fixed2),
            pl.BlockSpec((1, _D), fixed2),
            pl.BlockSpec((_D, _D), fixed2),
            pl.BlockSpec((1, _D), fixed2),
        ],
        out_specs=[
            pl.BlockSpec((_BLK, _D), row),
            pl.BlockSpec((_BLK, _D), row),
        ],
        out_shape=[
            jax.ShapeDtypeStruct((_NP, _D), _f32),
            jax.ShapeDtypeStruct((_NP, _D), _f32),
        ],
    )(h, s_parts, r_parts, w_rbf, v_msg, b_rbf, b_msg,
      wd, w1h, w1a, b1, g1, be1, w2, b2, g2, be2, wn, bn)


def kernel(x, pos, edge_index, W_msg, V_msg, b_msg, W_rbf, b_rbf,
           W_u1, b_u1, g1, be1, W_u2, b_u2, g2, be2, W_out, b_out):
    src = edge_index[0].astype(_i32)
    dst = edge_index[1].astype(_i32)
    pad = _N + (jnp.arange(_EPAD - _E, dtype=_i32) % (_NP - _N))
    src2 = jnp.concatenate([src, pad]).reshape(_EPAD // _CH, _CH)
    dst2 = jnp.concatenate([dst, pad]).reshape(_EPAD // _CH, _CH)
    src2r = src2.reshape(_EPAD // _RCH, _RCH)
    dst2r = dst2.reshape(_EPAD // _RCH, _RCH)
    zpad = jnp.zeros((_NP - _N,), _f32)
    px = jnp.concatenate([pos[:, 0], zpad])
    py = jnp.concatenate([pos[:, 1], zpad])
    pz = jnp.concatenate([pos[:, 2], zpad])
    xp = jnp.zeros((_NP, _D), _f32).at[:_N].set(x)

    r_parts = _sc_rbf_deg(px, py, pz, src2r, dst2r)

    def row2(b):
        return b.reshape(1, _D)

    b0 = _tc_mm(xp, W_msg[0, _D:])
    s_parts = _sc_scatter_rows(b0, src2, dst2)
    h1, b1t = _tc_update(
        xp, s_parts, r_parts, W_rbf[0], V_msg[0], row2(b_rbf[0]),
        row2(b_msg[0]), W_msg[0, :_D], W_u1[0, :_D], W_u1[0, _D:],
        row2(b_u1[0]), row2(g1[0]), row2(be1[0]), W_u2[0], row2(b_u2[0]),
        row2(g2[0]), row2(be2[0]), W_msg[1, _D:], jnp.zeros((1, _D), _f32))
    s_parts1 = _sc_scatter_rows(b1t, src2, dst2)
    _, out = _tc_update(
        h1, s_parts1, r_parts, W_rbf[1], V_msg[1], row2(b_rbf[1]),
        row2(b_msg[1]), W_msg[1, :_D], W_u1[1, :_D], W_u1[1, _D:],
        row2(b_u1[1]), row2(g1[1]), row2(be1[1]), W_u2[1], row2(b_u2[1]),
        row2(g2[1]), row2(be2[1]), W_out, row2(b_out))
    return out[:_N]


# speedup vs baseline: 13.2048x; 1.0017x over previous
import functools

import jax
import jax.numpy as jnp
from jax import lax
from jax.experimental import pallas as pl
from jax.experimental.pallas import tpu as pltpu
from jax.experimental.pallas import tpu_sc as plsc

_N = 10000
_E = 320000
_D = 128
_K = 64
_RMAX = 30.0
_NP = 10112
_RW = 128
_CH = 128
_CPT = 80
_EPAD = 32 * _CPT * _CH
_RPT = _NP // 16

_f32 = jnp.float32
_i32 = jnp.int32

_MESH = plsc.VectorSubcoreMesh(core_axis_name="c", subcore_axis_name="s")


def _zero_fill(zbuf, rows, width):
    zv = jnp.zeros((16,), _f32)
    for i in range(rows):
        for j in range(width // 16):
            zbuf[i, pl.ds(16 * j, 16)] = zv


_IBLK = 8
_RCH = 64
_RCPT = _EPAD // (32 * _RCH)


@functools.partial(
    pl.kernel,
    out_type=jax.ShapeDtypeStruct((2, _NP, _RW), _f32),
    mesh=_MESH,
    scratch_types=[
        pltpu.VMEM((_NP,), _f32),
        pltpu.VMEM((_NP,), _f32),
        pltpu.VMEM((_NP,), _f32),
        pltpu.VMEM((_IBLK, _RCH), _i32),
        pltpu.VMEM((_IBLK, _RCH), _i32),
        pltpu.VMEM((_RCH, _RW), _f32),
        pltpu.VMEM((_RCH, _RW), _f32),
        pltpu.VMEM((8, _RW), _f32),
        pltpu.VMEM_SHARED((_NP, _RW), _f32),
        pltpu.SemaphoreType.DMA,
        pltpu.SemaphoreType.DMA,
    ],
    compiler_params=pltpu.CompilerParams(needs_layout_passes=False),
)
def _sc_rbf_deg(px_hbm, py_hbm, pz_hbm, src_hbm, dst_hbm, out_hbm,
                px, py, pz, srcb, dstb, rows0, rows1, zbuf, acc,
                sem0, sem1):
    c = lax.axis_index("c")
    s = lax.axis_index("s")
    wid = s * 2 + c

    pltpu.sync_copy(px_hbm, px)
    pltpu.sync_copy(py_hbm, py)
    pltpu.sync_copy(pz_hbm, pz)

    _zero_fill(zbuf, 8, _RW)

    def zbody(i, carry):
        pltpu.sync_copy(zbuf, acc.at[pl.ds(s * _RPT + i * 8, 8)])
        return carry

    lax.fori_loop(0, _RPT // 8, zbody, 0)

    lane_i = lax.iota(_i32, 16)
    lane_f = lane_i.astype(_f32)
    inv_w = float(_K) / _RMAX
    centers_s = [(lane_f + (16.0 * g)) * (_RMAX / (_K - 1) * inv_w)
                 for g in range(4)]
    neg_half = -0.5
    zvec = jnp.zeros((16,), _f32)
    deg_one = jnp.where(lane_i == 0, 1.0, 0.0).astype(_f32)

    bufs = (rows0, rows1)
    sems = (sem0, sem1)

    for rbuf in bufs:
        def const_cols(r, carry, rbuf=rbuf):
            rbuf[r, pl.ds(64, 16)] = deg_one
            rbuf[r, pl.ds(80, 16)] = zvec
            rbuf[r, pl.ds(96, 16)] = zvec
            rbuf[r, pl.ds(112, 16)] = zvec
            return carry

        lax.fori_loop(0, _RCH, const_cols, 0)
    plsc.subcore_barrier()

    def blk_body(bi, carry):
        base = pl.multiple_of(wid * _RCPT + bi * _IBLK, _IBLK)
        pltpu.sync_copy(src_hbm.at[pl.ds(base, _IBLK)], srcb)
        pltpu.sync_copy(dst_hbm.at[pl.ds(base, _IBLK)], dstb)

        for cj in range(_IBLK):
            b = cj % 2
            rows = bufs[b]
            if cj >= 2:
                pltpu.make_async_copy(
                    rows, acc.at[dstb.at[cj]], sems[b]).wait()

            def group_body(g, gcarry, rows=rows, cj=cj):
                off = pl.multiple_of(16 * g, 16)
                si = srcb[cj, pl.ds(off, 16)]
                di = dstb[cj, pl.ds(off, 16)]
                dx = plsc.load_gather(px, [di]) - plsc.load_gather(px, [si])
                dy = plsc.load_gather(py, [di]) - plsc.load_gather(py, [si])
                dz = plsc.load_gather(pz, [di]) - plsc.load_gather(pz, [si])
                dd = dx * dx + dy * dy + dz * dz + 1e-12
                bits = lax.bitcast_convert_type(dd, _i32)
                y = lax.bitcast_convert_type(
                    lax.shift_right_logical(bits, 1) + 0x1FBD1DF6, _f32)
                for _ in range(3):
                    y = 0.5 * (y + dd / y)
                ys = y * inv_w
                for el in range(16):
                    dv = ys.at[jnp.full((16,), el, _i32)].get(
                        mode="promise_in_bounds")
                    for j in range(4):
                        t = dv - centers_s[j]
                        rows[off + el, pl.ds(16 * j, 16)] = (
                            jnp.exp((t * t) * neg_half))
                return gcarry

            lax.fori_loop(0, _RCH // 16, group_body, 0)
            pltpu.async_copy(rows, acc.at[dstb.at[cj]], sems[b], add=True)

        pltpu.make_async_copy(bufs[0], acc.at[dstb.at[_IBLK - 2]],
                              sems[0]).wait()
        pltpu.make_async_copy(bufs[1], acc.at[dstb.at[_IBLK - 1]],
                              sems[1]).wait()
        return carry

    lax.fori_loop(0, _RCPT // _IBLK, blk_body, 0)
    plsc.subcore_barrier()
    pltpu.sync_copy(acc.at[pl.ds(s * _RPT, _RPT)],
                    out_hbm.at[c, pl.ds(s * _RPT, _RPT)])


_SBLK = 8


@functools.partial(
    pl.kernel,
    out_type=jax.ShapeDtypeStruct((2, _NP, _D), _f32),
    mesh=_MESH,
    scratch_types=[
        pltpu.VMEM((_SBLK, _CH), _i32),
        pltpu.VMEM((_SBLK, _CH), _i32),
        pltpu.VMEM((_CH, _D), _f32),
        pltpu.VMEM((_CH, _D), _f32),
        pltpu.VMEM((8, _D), _f32),
        pltpu.VMEM_SHARED((_NP, _D), _f32),
        pltpu.SemaphoreType.DMA,
        pltpu.SemaphoreType.DMA,
        pltpu.SemaphoreType.DMA,
        pltpu.SemaphoreType.DMA,
    ],
    compiler_params=pltpu.CompilerParams(needs_layout_passes=False),
)
def _sc_scatter_rows(tbl_hbm, src_hbm, dst_hbm, out_hbm,
                     srcb, dstb, rows0, rows1, zbuf, acc,
                     gsem0, gsem1, ssem0, ssem1):
    c = lax.axis_index("c")
    s = lax.axis_index("s")
    wid = s * 2 + c

    _zero_fill(zbuf, 8, _D)

    def zbody(i, carry):
        pltpu.sync_copy(zbuf, acc.at[pl.ds(s * _RPT + i * 8, 8)])
        return carry

    lax.fori_loop(0, _RPT // 8, zbody, 0)
    plsc.subcore_barrier()

    bufs = (rows0, rows1)
    gsems = (gsem0, gsem1)
    ssems = (ssem0, ssem1)

    def blk_body(bi, carry):
        base = pl.multiple_of(wid * _CPT + bi * _SBLK, _SBLK)
        pltpu.sync_copy(src_hbm.at[pl.ds(base, _SBLK)], srcb)
        pltpu.sync_copy(dst_hbm.at[pl.ds(base, _SBLK)], dstb)
        pltpu.async_copy(tbl_hbm.at[srcb.at[0]], bufs[0], gsems[0])
        for cj in range(_SBLK):
            b = cj % 2
            nb = (cj + 1) % 2
            if cj + 1 < _SBLK:
                if cj >= 1:
                    pltpu.make_async_copy(
                        bufs[nb], acc.at[dstb.at[cj - 1]], ssems[nb]).wait()
                pltpu.async_copy(
                    tbl_hbm.at[srcb.at[cj + 1]], bufs[nb], gsems[nb])
            pltpu.make_async_copy(
                tbl_hbm.at[srcb.at[cj]], bufs[b], gsems[b]).wait()
            pltpu.async_copy(bufs[b], acc.at[dstb.at[cj]], ssems[b],
                             add=True)
        pltpu.make_async_copy(bufs[0], acc.at[dstb.at[_SBLK - 2]],
                              ssems[0]).wait()
        pltpu.make_async_copy(bufs[1], acc.at[dstb.at[_SBLK - 1]],
                              ssems[1]).wait()
        return carry

    lax.fori_loop(0, _CPT // _SBLK, blk_body, 0)
    plsc.subcore_barrier()
    pltpu.sync_copy(acc.at[pl.ds(s * _RPT, _RPT)],
                    out_hbm.at[c, pl.ds(s * _RPT, _RPT)])


_BLK = 1264


def _mm_body(x_ref, w_ref, o_ref):
    o_ref[...] = jnp.dot(x_ref[...], w_ref[...],
                         preferred_element_type=_f32)


def _tc_mm(x, w):
    return pl.pallas_call(
        _mm_body,
        grid=(_NP // _BLK,),
        in_specs=[
            pl.BlockSpec((_BLK, _D), lambda i: (i, 0)),
            pl.BlockSpec((_D, _D), lambda i: (0, 0)),
        ],
        out_specs=pl.BlockSpec((_BLK, _D), lambda i: (i, 0)),
        out_shape=jax.ShapeDtypeStruct((_NP, _D), _f32),
    )(x, w)


def _ln_relu(u, g, b):
    mu = jnp.mean(u, axis=-1, keepdims=True)
    d = u - mu
    var = jnp.mean(d * d, axis=-1, keepdims=True)
    z = d / jnp.sqrt(var + 1e-5) * g + b
    return jnp.maximum(z, 0.0)


def _upd_body(h_ref, sp_ref, rp_ref, wr_ref, v_ref, brbf_ref, bmsg_ref,
              wd_ref, w1h_ref, w1a_ref, b1_ref, g1_ref, be1_ref,
              w2_ref, b2_ref, g2_ref, be2_ref, wn_ref, bn_ref,
              hn_ref, yn_ref):
    dot = functools.partial(jnp.dot, preferred_element_type=_f32)
    sagg = sp_ref[0] + sp_ref[1]
    rq = rp_ref[0] + rp_ref[1]
    rm = rq[:, :_K]
    deg = rq[:, _K:_K + 1]
    wrv = dot(wr_ref[...], v_ref[...])
    cvec = dot(brbf_ref[...], v_ref[...]) + bmsg_ref[...]
    h = h_ref[...]
    a = dot(h, wd_ref[...])
    aggr = sagg + deg * a + deg * cvec + dot(rm, wrv)
    u = dot(h, w1h_ref[...]) + dot(aggr, w1a_ref[...]) + b1_ref[...]
    z = _ln_relu(u, g1_ref[...], be1_ref[...])
    z = _ln_relu(dot(z, w2_ref[...]) + b2_ref[...], g2_ref[...], be2_ref[...])
    hn = jnp.where(deg > 0.0, h + z, h)
    hn_ref[...] = hn
    yn_ref[...] = dot(hn, wn_ref[...]) + bn_ref[...]


def _tc_update(h, s_parts, r_parts, w_rbf, v_msg, b_rbf, b_msg,
               wd, w1h, w1a, b1, g1, be1, w2, b2, g2, be2, wn, bn):
    row = lambda i: (i, 0)
    part = lambda i: (0, i, 0)
    fixed2 = lambda i: (0, 0)
    return pl.pallas_call(
        _upd_body,
        grid=(_NP // _BLK,),
        in_specs=[
            pl.BlockSpec((_BLK, _D), row),
            pl.BlockSpec((2, _BLK, _D), part),
            pl.BlockSpec((2, _BLK, _RW), part),
            pl.BlockSpec((_K, _D), fixed2),
            pl.BlockSpec((_D, _D), fixed2),
            pl.BlockSpec((1, _D), fixed2),
            pl.BlockSpec((1, _D), fixed2),
            pl.BlockSpec((_D, _D), fixed2),
            pl.BlockSpec((_D, _D), fixed2),
            pl.BlockSpec((_D, _D), fixed2),
            pl.BlockSpec((1, _D), fixed2),
            pl.BlockSpec((1, _D), fixed2),
            pl.BlockSpec((1, _D), fixed2),
            pl.BlockSpec((_D, _D), fixed2),
            pl.BlockSpec((1, _D), fixed2),
            pl.BlockSpec((1, _D), fixed2),
            pl.BlockSpec((1, _D), fixed2),
            pl.BlockSpec((_D, _D), fixed2),
            pl.BlockSpec((1, _D), fixed2),
        ],
        out_specs=[
            pl.BlockSpec((_BLK, _D), row),
            pl.BlockSpec((_BLK, _D), row),
        ],
        out_shape=[
            jax.ShapeDtypeStruct((_NP, _D), _f32),
            jax.ShapeDtypeStruct((_NP, _D), _f32),
        ],
    )(h, s_parts, r_parts, w_rbf, v_msg, b_rbf, b_msg,
      wd, w1h, w1a, b1, g1, be1, w2, b2, g2, be2, wn, bn)


def kernel(x, pos, edge_index, W_msg, V_msg, b_msg, W_rbf, b_rbf,
           W_u1, b_u1, g1, be1, W_u2, b_u2, g2, be2, W_out, b_out):
    src = edge_index[0].astype(_i32)
    dst = edge_index[1].astype(_i32)
    pad = _N + (jnp.arange(_EPAD - _E, dtype=_i32) % (_NP - _N))
    src2 = jnp.concatenate([src, pad]).reshape(_EPAD // _CH, _CH)
    dst2 = jnp.concatenate([dst, pad]).reshape(_EPAD // _CH, _CH)
    src2r = src2.reshape(_EPAD // _RCH, _RCH)
    dst2r = dst2.reshape(_EPAD // _RCH, _RCH)
    zpad = jnp.zeros((_NP - _N,), _f32)
    px = jnp.concatenate([pos[:, 0], zpad])
    py = jnp.concatenate([pos[:, 1], zpad])
    pz = jnp.concatenate([pos[:, 2], zpad])
    xp = jnp.zeros((_NP, _D), _f32).at[:_N].set(x)

    r_parts = _sc_rbf_deg(px, py, pz, src2r, dst2r)

    def row2(b):
        return b.reshape(1, _D)

    b0 = _tc_mm(xp, W_msg[0, _D:])
    s_parts = _sc_scatter_rows(b0, src2, dst2)
    h1, b1t = _tc_update(
        xp, s_parts, r_parts, W_rbf[0], V_msg[0], row2(b_rbf[0]),
        row2(b_msg[0]), W_msg[0, :_D], W_u1[0, :_D], W_u1[0, _D:],
        row2(b_u1[0]), row2(g1[0]), row2(be1[0]), W_u2[0], row2(b_u2[0]),
        row2(g2[0]), row2(be2[0]), W_msg[1, _D:], jnp.zeros((1, _D), _f32))
    s_parts1 = _sc_scatter_rows(b1t, src2, dst2)
    _, out = _tc_update(
        h1, s_parts1, r_parts, W_rbf[1], V_msg[1], row2(b_rbf[1]),
        row2(b_msg[1]), W_msg[1, :_D], W_u1[1, :_D], W_u1[1, _D:],
        row2(b_u1[1]), row2(g1[1]), row2(be1[1]), W_u2[1], row2(b_u2[1]),
        row2(g2[1]), row2(be2[1]), W_out, row2(b_out))
    return out[:_N]


# speedup vs baseline: 13.5123x; 1.0233x over previous
import functools

import jax
import jax.numpy as jnp
from jax import lax
from jax.experimental import pallas as pl
from jax.experimental.pallas import tpu as pltpu
from jax.experimental.pallas import tpu_sc as plsc

_N = 10000
_E = 320000
_D = 128
_K = 64
_RMAX = 30.0
_NP = 10112
_RW = 128
_CH = 128
_CPT = 80
_EPAD = 32 * _CPT * _CH
_RPT = _NP // 16

_f32 = jnp.float32
_i32 = jnp.int32

_MESH = plsc.VectorSubcoreMesh(core_axis_name="c", subcore_axis_name="s")


def _zero_fill(zbuf, rows, width):
    zv = jnp.zeros((16,), _f32)
    for i in range(rows):
        for j in range(width // 16):
            zbuf[i, pl.ds(16 * j, 16)] = zv


_IBLK = 8
_RCH = 64
_RCPT = _EPAD // (32 * _RCH)


@functools.partial(
    pl.kernel,
    out_type=jax.ShapeDtypeStruct((2, _NP, _RW), _f32),
    mesh=_MESH,
    scratch_types=[
        pltpu.VMEM((_NP,), _f32),
        pltpu.VMEM((_NP,), _f32),
        pltpu.VMEM((_NP,), _f32),
        pltpu.VMEM((_IBLK, _RCH), _i32),
        pltpu.VMEM((_IBLK, _RCH), _i32),
        pltpu.VMEM((_RCH, _RW), _f32),
        pltpu.VMEM((_RCH, _RW), _f32),
        pltpu.VMEM((8, _RW), _f32),
        pltpu.VMEM_SHARED((_NP, _RW), _f32),
        pltpu.SemaphoreType.DMA,
        pltpu.SemaphoreType.DMA,
    ],
    compiler_params=pltpu.CompilerParams(needs_layout_passes=False),
)
def _sc_rbf_deg(px_hbm, py_hbm, pz_hbm, src_hbm, dst_hbm, out_hbm,
                px, py, pz, srcb, dstb, rows0, rows1, zbuf, acc,
                sem0, sem1):
    c = lax.axis_index("c")
    s = lax.axis_index("s")
    wid = s * 2 + c

    pltpu.sync_copy(px_hbm, px)
    pltpu.sync_copy(py_hbm, py)
    pltpu.sync_copy(pz_hbm, pz)

    _zero_fill(zbuf, 8, _RW)

    def zbody(i, carry):
        pltpu.sync_copy(zbuf, acc.at[pl.ds(s * _RPT + i * 8, 8)])
        return carry

    lax.fori_loop(0, _RPT // 8, zbody, 0)

    lane_i = lax.iota(_i32, 16)
    lane_f = lane_i.astype(_f32)
    inv_w = float(_K) / _RMAX
    centers_s = [(lane_f + (16.0 * g)) * (_RMAX / (_K - 1) * inv_w)
                 for g in range(4)]
    neg_half = -0.5
    zvec = jnp.zeros((16,), _f32)
    deg_one = jnp.where(lane_i == 0, 1.0, 0.0).astype(_f32)

    bufs = (rows0, rows1)
    sems = (sem0, sem1)

    for rbuf in bufs:
        def const_cols(r, carry, rbuf=rbuf):
            rbuf[r, pl.ds(64, 16)] = deg_one
            rbuf[r, pl.ds(80, 16)] = zvec
            rbuf[r, pl.ds(96, 16)] = zvec
            rbuf[r, pl.ds(112, 16)] = zvec
            return carry

        lax.fori_loop(0, _RCH, const_cols, 0)
    plsc.subcore_barrier()

    def blk_body(bi, carry):
        base = pl.multiple_of(wid * _RCPT + bi * _IBLK, _IBLK)
        pltpu.sync_copy(src_hbm.at[pl.ds(base, _IBLK)], srcb)
        pltpu.sync_copy(dst_hbm.at[pl.ds(base, _IBLK)], dstb)

        for cj in range(_IBLK):
            b = cj % 2
            rows = bufs[b]
            if cj >= 2:
                pltpu.make_async_copy(
                    rows, acc.at[dstb.at[cj]], sems[b]).wait()

            def group_body(g, gcarry, rows=rows, cj=cj):
                off = pl.multiple_of(16 * g, 16)
                si = srcb[cj, pl.ds(off, 16)]
                di = dstb[cj, pl.ds(off, 16)]
                dx = plsc.load_gather(px, [di]) - plsc.load_gather(px, [si])
                dy = plsc.load_gather(py, [di]) - plsc.load_gather(py, [si])
                dz = plsc.load_gather(pz, [di]) - plsc.load_gather(pz, [si])
                dd = dx * dx + dy * dy + dz * dz + 1e-12
                bits = lax.bitcast_convert_type(dd, _i32)
                y = lax.bitcast_convert_type(
                    lax.shift_right_logical(bits, 1) + 0x1FBD1DF6, _f32)
                for _ in range(3):
                    y = 0.5 * (y + dd / y)
                ys = y * inv_w
                for el in range(16):
                    dv = ys.at[jnp.full((16,), el, _i32)].get(
                        mode="promise_in_bounds")
                    for j in range(4):
                        t = dv - centers_s[j]
                        rows[off + el, pl.ds(16 * j, 16)] = (
                            jnp.exp((t * t) * neg_half))
                return gcarry

            lax.fori_loop(0, _RCH // 16, group_body, 0)
            pltpu.async_copy(rows, acc.at[dstb.at[cj]], sems[b], add=True)

        pltpu.make_async_copy(bufs[0], acc.at[dstb.at[_IBLK - 2]],
                              sems[0]).wait()
        pltpu.make_async_copy(bufs[1], acc.at[dstb.at[_IBLK - 1]],
                              sems[1]).wait()
        return carry

    lax.fori_loop(0, _RCPT // _IBLK, blk_body, 0)
    plsc.subcore_barrier()
    pltpu.sync_copy(acc.at[pl.ds(s * _RPT, _RPT)],
                    out_hbm.at[c, pl.ds(s * _RPT, _RPT)])


_SBLK = 8


@functools.partial(
    pl.kernel,
    out_type=jax.ShapeDtypeStruct((2, _NP, _D), _f32),
    mesh=_MESH,
    scratch_types=[
        pltpu.VMEM((_SBLK, _CH), _i32),
        pltpu.VMEM((_SBLK, _CH), _i32),
        pltpu.VMEM((_CH, _D), _f32),
        pltpu.VMEM((_CH, _D), _f32),
        pltpu.VMEM((8, _D), _f32),
        pltpu.VMEM_SHARED((_NP, _D), _f32),
        pltpu.SemaphoreType.DMA,
        pltpu.SemaphoreType.DMA,
        pltpu.SemaphoreType.DMA,
        pltpu.SemaphoreType.DMA,
    ],
    compiler_params=pltpu.CompilerParams(needs_layout_passes=False),
)
def _sc_scatter_rows(tbl_hbm, src_hbm, dst_hbm, out_hbm,
                     srcb, dstb, rows0, rows1, zbuf, acc,
                     gsem0, gsem1, ssem0, ssem1):
    c = lax.axis_index("c")
    s = lax.axis_index("s")
    wid = s * 2 + c

    _zero_fill(zbuf, 8, _D)

    def zbody(i, carry):
        pltpu.sync_copy(zbuf, acc.at[pl.ds(s * _RPT + i * 8, 8)])
        return carry

    lax.fori_loop(0, _RPT // 8, zbody, 0)
    plsc.subcore_barrier()

    bufs = (rows0, rows1)
    gsems = (gsem0, gsem1)
    ssems = (ssem0, ssem1)

    def blk_body(bi, carry):
        base = pl.multiple_of(wid * _CPT + bi * _SBLK, _SBLK)
        pltpu.sync_copy(src_hbm.at[pl.ds(base, _SBLK)], srcb)
        pltpu.sync_copy(dst_hbm.at[pl.ds(base, _SBLK)], dstb)
        pltpu.async_copy(tbl_hbm.at[srcb.at[0]], bufs[0], gsems[0])
        for cj in range(_SBLK):
            b = cj % 2
            nb = (cj + 1) % 2
            if cj + 1 < _SBLK:
                if cj >= 1:
                    pltpu.make_async_copy(
                        bufs[nb], acc.at[dstb.at[cj - 1]], ssems[nb]).wait()
                pltpu.async_copy(
                    tbl_hbm.at[srcb.at[cj + 1]], bufs[nb], gsems[nb])
            pltpu.make_async_copy(
                tbl_hbm.at[srcb.at[cj]], bufs[b], gsems[b]).wait()
            pltpu.async_copy(bufs[b], acc.at[dstb.at[cj]], ssems[b],
                             add=True)
        pltpu.make_async_copy(bufs[0], acc.at[dstb.at[_SBLK - 2]],
                              ssems[0]).wait()
        pltpu.make_async_copy(bufs[1], acc.at[dstb.at[_SBLK - 1]],
                              ssems[1]).wait()
        return carry

    lax.fori_loop(0, _CPT // _SBLK, blk_body, 0)
    plsc.subcore_barrier()
    pltpu.sync_copy(acc.at[pl.ds(s * _RPT, _RPT)],
                    out_hbm.at[c, pl.ds(s * _RPT, _RPT)])


_BLK = 1264


def _ln_relu(u, g, b):
    mu = jnp.mean(u, axis=-1, keepdims=True)
    d = u - mu
    var = jnp.mean(d * d, axis=-1, keepdims=True)
    z = d / jnp.sqrt(var + 1e-5) * g + b
    return jnp.maximum(z, 0.0)


def _make_upd_body(tail):
    def body(h_ref, sp_ref, rp_ref, ws_ref, wr_ref, v_ref, brbf_ref,
             bmsg_ref, wd_ref, w1h_ref, w1a_ref, b1_ref, g1_ref, be1_ref,
             w2_ref, b2_ref, g2_ref, be2_ref, *refs):
        dot = functools.partial(jnp.dot, preferred_element_type=_f32)
        if tail:
            wn_ref, bn_ref, hn_ref, yn_ref = refs
        else:
            (hn_ref,) = refs
        sagg = dot(sp_ref[0] + sp_ref[1], ws_ref[...])
        rq = rp_ref[0] + rp_ref[1]
        rm = rq[:, :_K]
        deg = rq[:, _K:_K + 1]
        wrv = dot(wr_ref[...], v_ref[...])
        cvec = dot(brbf_ref[...], v_ref[...]) + bmsg_ref[...]
        h = h_ref[...]
        a = dot(h, wd_ref[...])
        aggr = sagg + deg * a + deg * cvec + dot(rm, wrv)
        u = dot(h, w1h_ref[...]) + dot(aggr, w1a_ref[...]) + b1_ref[...]
        z = _ln_relu(u, g1_ref[...], be1_ref[...])
        z = _ln_relu(dot(z, w2_ref[...]) + b2_ref[...],
                     g2_ref[...], be2_ref[...])
        hn = jnp.where(deg > 0.0, h + z, h)
        hn_ref[...] = hn
        if tail:
            yn_ref[...] = dot(hn, wn_ref[...]) + bn_ref[...]

    return body


def _tc_update(h, s_parts, r_parts, ws, w_rbf, v_msg, b_rbf, b_msg,
               wd, w1h, w1a, b1, g1, be1, w2, b2, g2, be2, wn=None, bn=None):
    row = lambda i: (i, 0)
    part = lambda i: (0, i, 0)
    fixed2 = lambda i: (0, 0)
    tail = wn is not None
    in_specs = [
        pl.BlockSpec((_BLK, _D), row),
        pl.BlockSpec((2, _BLK, _D), part),
        pl.BlockSpec((2, _BLK, _RW), part),
        pl.BlockSpec((_D, _D), fixed2),
        pl.BlockSpec((_K, _D), fixed2),
        pl.BlockSpec((_D, _D), fixed2),
        pl.BlockSpec((1, _D), fixed2),
        pl.BlockSpec((1, _D), fixed2),
        pl.BlockSpec((_D, _D), fixed2),
        pl.BlockSpec((_D, _D), fixed2),
        pl.BlockSpec((_D, _D), fixed2),
        pl.BlockSpec((1, _D), fixed2),
        pl.BlockSpec((1, _D), fixed2),
        pl.BlockSpec((1, _D), fixed2),
        pl.BlockSpec((_D, _D), fixed2),
        pl.BlockSpec((1, _D), fixed2),
        pl.BlockSpec((1, _D), fixed2),
        pl.BlockSpec((1, _D), fixed2),
    ]
    args = [h, s_parts, r_parts, ws, w_rbf, v_msg, b_rbf, b_msg,
            wd, w1h, w1a, b1, g1, be1, w2, b2, g2, be2]
    out_specs = [pl.BlockSpec((_BLK, _D), row)]
    out_shape = [jax.ShapeDtypeStruct((_NP, _D), _f32)]
    if tail:
        in_specs += [pl.BlockSpec((_D, _D), fixed2),
                     pl.BlockSpec((1, _D), fixed2)]
        args += [wn, bn]
        out_specs.append(pl.BlockSpec((_BLK, _D), row))
        out_shape.append(jax.ShapeDtypeStruct((_NP, _D), _f32))
    res = pl.pallas_call(
        _make_upd_body(tail),
        grid=(_NP // _BLK,),
        in_specs=in_specs,
        out_specs=out_specs,
        out_shape=out_shape,
    )(*args)
    return res if tail else res[0]


def kernel(x, pos, edge_index, W_msg, V_msg, b_msg, W_rbf, b_rbf,
           W_u1, b_u1, g1, be1, W_u2, b_u2, g2, be2, W_out, b_out):
    src = edge_index[0].astype(_i32)
    dst = edge_index[1].astype(_i32)
    pad = _N + (jnp.arange(_EPAD - _E, dtype=_i32) % (_NP - _N))
    src2 = jnp.concatenate([src, pad]).reshape(_EPAD // _CH, _CH)
    dst2 = jnp.concatenate([dst, pad]).reshape(_EPAD // _CH, _CH)
    src2r = src2.reshape(_EPAD // _RCH, _RCH)
    dst2r = dst2.reshape(_EPAD // _RCH, _RCH)
    zpad = jnp.zeros((_NP - _N,), _f32)
    px = jnp.concatenate([pos[:, 0], zpad])
    py = jnp.concatenate([pos[:, 1], zpad])
    pz = jnp.concatenate([pos[:, 2], zpad])
    xp = jnp.zeros((_NP, _D), _f32).at[:_N].set(x)

    r_parts = _sc_rbf_deg(px, py, pz, src2r, dst2r)

    def row2(b):
        return b.reshape(1, _D)

    s_parts = _sc_scatter_rows(xp, src2, dst2)
    h1 = _tc_update(
        xp, s_parts, r_parts, W_msg[0, _D:], W_rbf[0], V_msg[0],
        row2(b_rbf[0]), row2(b_msg[0]), W_msg[0, :_D], W_u1[0, :_D],
        W_u1[0, _D:], row2(b_u1[0]), row2(g1[0]), row2(be1[0]), W_u2[0],
        row2(b_u2[0]), row2(g2[0]), row2(be2[0]))
    s_parts1 = _sc_scatter_rows(h1, src2, dst2)
    _, out = _tc_update(
        h1, s_parts1, r_parts, W_msg[1, _D:], W_rbf[1], V_msg[1],
        row2(b_rbf[1]), row2(b_msg[1]), W_msg[1, :_D], W_u1[1, :_D],
        W_u1[1, _D:], row2(b_u1[1]), row2(g1[1]), row2(be1[1]), W_u2[1],
        row2(b_u2[1]), row2(g2[1]), row2(be2[1]), W_out, row2(b_out))
    return out[:_N]


# speedup vs baseline: 13.8468x; 1.0248x over previous
import functools

import jax
import jax.numpy as jnp
from jax import lax
from jax.experimental import pallas as pl
from jax.experimental.pallas import tpu as pltpu
from jax.experimental.pallas import tpu_sc as plsc

_N = 10000
_E = 320000
_D = 128
_K = 64
_RMAX = 30.0
_NP = 10112
_RW = 128
_CH = 128
_CPT = 80
_EPAD = 32 * _CPT * _CH
_RPT = _NP // 16

_f32 = jnp.float32
_i32 = jnp.int32

_MESH = plsc.VectorSubcoreMesh(core_axis_name="c", subcore_axis_name="s")


def _zero_fill(zbuf, rows, width):
    zv = jnp.zeros((16,), _f32)
    for i in range(rows):
        for j in range(width // 16):
            zbuf[i, pl.ds(16 * j, 16)] = zv


def _zero_acc(zbuf, acc, s, sem):
    def issue(i, carry):
        pltpu.async_copy(zbuf, acc.at[pl.ds(s * _RPT + i * 8, 8)], sem)
        return carry

    lax.fori_loop(0, _RPT // 8, issue, 0)

    def drain(i, carry):
        pltpu.make_async_copy(zbuf, acc.at[pl.ds(s * _RPT, 8)], sem).wait()
        return carry

    lax.fori_loop(0, _RPT // 8, drain, 0)


_IBLK = 8
_RCH = 64
_RCPT = _EPAD // (32 * _RCH)


@functools.partial(
    pl.kernel,
    out_type=jax.ShapeDtypeStruct((2, _NP, _RW), _f32),
    mesh=_MESH,
    scratch_types=[
        pltpu.VMEM((_NP,), _f32),
        pltpu.VMEM((_NP,), _f32),
        pltpu.VMEM((_NP,), _f32),
        pltpu.VMEM((_IBLK, _RCH), _i32),
        pltpu.VMEM((_IBLK, _RCH), _i32),
        pltpu.VMEM((_RCH, _RW), _f32),
        pltpu.VMEM((_RCH, _RW), _f32),
        pltpu.VMEM((8, _RW), _f32),
        pltpu.VMEM_SHARED((_NP, _RW), _f32),
        pltpu.SemaphoreType.DMA,
        pltpu.SemaphoreType.DMA,
    ],
    compiler_params=pltpu.CompilerParams(needs_layout_passes=False),
)
def _sc_rbf_deg(px_hbm, py_hbm, pz_hbm, src_hbm, dst_hbm, out_hbm,
                px, py, pz, srcb, dstb, rows0, rows1, zbuf, acc,
                sem0, sem1):
    c = lax.axis_index("c")
    s = lax.axis_index("s")
    wid = s * 2 + c

    pltpu.sync_copy(px_hbm, px)
    pltpu.sync_copy(py_hbm, py)
    pltpu.sync_copy(pz_hbm, pz)

    _zero_fill(zbuf, 8, _RW)
    _zero_acc(zbuf, acc, s, sem0)

    lane_i = lax.iota(_i32, 16)
    lane_f = lane_i.astype(_f32)
    inv_w = float(_K) / _RMAX
    centers_s = [(lane_f + (16.0 * g)) * (_RMAX / (_K - 1) * inv_w)
                 for g in range(4)]
    neg_half = -0.5
    zvec = jnp.zeros((16,), _f32)
    deg_one = jnp.where(lane_i == 0, 1.0, 0.0).astype(_f32)

    bufs = (rows0, rows1)
    sems = (sem0, sem1)

    for rbuf in bufs:
        def const_cols(r, carry, rbuf=rbuf):
            rbuf[r, pl.ds(64, 16)] = deg_one
            rbuf[r, pl.ds(80, 16)] = zvec
            rbuf[r, pl.ds(96, 16)] = zvec
            rbuf[r, pl.ds(112, 16)] = zvec
            return carry

        lax.fori_loop(0, _RCH, const_cols, 0)
    plsc.subcore_barrier()

    def blk_body(bi, carry):
        base = pl.multiple_of(wid * _RCPT + bi * _IBLK, _IBLK)
        pltpu.sync_copy(src_hbm.at[pl.ds(base, _IBLK)], srcb)
        pltpu.sync_copy(dst_hbm.at[pl.ds(base, _IBLK)], dstb)

        for cj in range(_IBLK):
            b = cj % 2
            rows = bufs[b]
            if cj >= 2:
                pltpu.make_async_copy(
                    rows, acc.at[dstb.at[cj]], sems[b]).wait()

            def group_body(g, gcarry, rows=rows, cj=cj):
                off = pl.multiple_of(16 * g, 16)
                si = srcb[cj, pl.ds(off, 16)]
                di = dstb[cj, pl.ds(off, 16)]
                dx = plsc.load_gather(px, [di]) - plsc.load_gather(px, [si])
                dy = plsc.load_gather(py, [di]) - plsc.load_gather(py, [si])
                dz = plsc.load_gather(pz, [di]) - plsc.load_gather(pz, [si])
                dd = dx * dx + dy * dy + dz * dz + 1e-12
                bits = lax.bitcast_convert_type(dd, _i32)
                y = lax.bitcast_convert_type(
                    lax.shift_right_logical(bits, 1) + 0x1FBD1DF6, _f32)
                for _ in range(3):
                    y = 0.5 * (y + dd / y)
                ys = y * inv_w
                for el in range(16):
                    dv = ys.at[jnp.full((16,), el, _i32)].get(
                        mode="promise_in_bounds")
                    for j in range(4):
                        t = dv - centers_s[j]
                        rows[off + el, pl.ds(16 * j, 16)] = (
                            jnp.exp((t * t) * neg_half))
                return gcarry

            lax.fori_loop(0, _RCH // 16, group_body, 0)
            pltpu.async_copy(rows, acc.at[dstb.at[cj]], sems[b], add=True)

        pltpu.make_async_copy(bufs[0], acc.at[dstb.at[_IBLK - 2]],
                              sems[0]).wait()
        pltpu.make_async_copy(bufs[1], acc.at[dstb.at[_IBLK - 1]],
                              sems[1]).wait()
        return carry

    lax.fori_loop(0, _RCPT // _IBLK, blk_body, 0)
    plsc.subcore_barrier()
    pltpu.sync_copy(acc.at[pl.ds(s * _RPT, _RPT)],
                    out_hbm.at[c, pl.ds(s * _RPT, _RPT)])


_SBLK = 8


@functools.partial(
    pl.kernel,
    out_type=jax.ShapeDtypeStruct((2, _NP, _D), _f32),
    mesh=_MESH,
    scratch_types=[
        pltpu.VMEM((_SBLK, _CH), _i32),
        pltpu.VMEM((_SBLK, _CH), _i32),
        pltpu.VMEM((_CH, _D), _f32),
        pltpu.VMEM((_CH, _D), _f32),
        pltpu.VMEM((8, _D), _f32),
        pltpu.VMEM_SHARED((_NP, _D), _f32),
        pltpu.SemaphoreType.DMA,
        pltpu.SemaphoreType.DMA,
        pltpu.SemaphoreType.DMA,
        pltpu.SemaphoreType.DMA,
    ],
    compiler_params=pltpu.CompilerParams(needs_layout_passes=False),
)
def _sc_scatter_rows(tbl_hbm, src_hbm, dst_hbm, out_hbm,
                     srcb, dstb, rows0, rows1, zbuf, acc,
                     gsem0, gsem1, ssem0, ssem1):
    c = lax.axis_index("c")
    s = lax.axis_index("s")
    wid = s * 2 + c

    _zero_fill(zbuf, 8, _D)
    _zero_acc(zbuf, acc, s, gsem0)
    plsc.subcore_barrier()

    bufs = (rows0, rows1)
    gsems = (gsem0, gsem1)
    ssems = (ssem0, ssem1)

    def blk_body(bi, carry):
        base = pl.multiple_of(wid * _CPT + bi * _SBLK, _SBLK)
        pltpu.sync_copy(src_hbm.at[pl.ds(base, _SBLK)], srcb)
        pltpu.sync_copy(dst_hbm.at[pl.ds(base, _SBLK)], dstb)
        pltpu.async_copy(tbl_hbm.at[srcb.at[0]], bufs[0], gsems[0])
        for cj in range(_SBLK):
            b = cj % 2
            nb = (cj + 1) % 2
            if cj + 1 < _SBLK:
                if cj >= 1:
                    pltpu.make_async_copy(
                        bufs[nb], acc.at[dstb.at[cj - 1]], ssems[nb]).wait()
                pltpu.async_copy(
                    tbl_hbm.at[srcb.at[cj + 1]], bufs[nb], gsems[nb])
            pltpu.make_async_copy(
                tbl_hbm.at[srcb.at[cj]], bufs[b], gsems[b]).wait()
            pltpu.async_copy(bufs[b], acc.at[dstb.at[cj]], ssems[b],
                             add=True)
        pltpu.make_async_copy(bufs[0], acc.at[dstb.at[_SBLK - 2]],
                              ssems[0]).wait()
        pltpu.make_async_copy(bufs[1], acc.at[dstb.at[_SBLK - 1]],
                              ssems[1]).wait()
        return carry

    lax.fori_loop(0, _CPT // _SBLK, blk_body, 0)
    plsc.subcore_barrier()
    pltpu.sync_copy(acc.at[pl.ds(s * _RPT, _RPT)],
                    out_hbm.at[c, pl.ds(s * _RPT, _RPT)])


_BLK = 1264


def _ln_relu(u, g, b):
    mu = jnp.mean(u, axis=-1, keepdims=True)
    d = u - mu
    var = jnp.mean(d * d, axis=-1, keepdims=True)
    z = d / jnp.sqrt(var + 1e-5) * g + b
    return jnp.maximum(z, 0.0)


def _make_upd_body(tail):
    def body(h_ref, sp_ref, rp_ref, ws_ref, wr_ref, v_ref, brbf_ref,
             bmsg_ref, wd_ref, w1h_ref, w1a_ref, b1_ref, g1_ref, be1_ref,
             w2_ref, b2_ref, g2_ref, be2_ref, *refs):
        dot = functools.partial(jnp.dot, preferred_element_type=_f32)
        if tail:
            wn_ref, bn_ref, hn_ref, yn_ref = refs
        else:
            (hn_ref,) = refs
        sagg = dot(sp_ref[0] + sp_ref[1], ws_ref[...])
        rq = rp_ref[0] + rp_ref[1]
        rm = rq[:, :_K]
        deg = rq[:, _K:_K + 1]
        wrv = dot(wr_ref[...], v_ref[...])
        cvec = dot(brbf_ref[...], v_ref[...]) + bmsg_ref[...]
        h = h_ref[...]
        a = dot(h, wd_ref[...])
        aggr = sagg + deg * a + deg * cvec + dot(rm, wrv)
        u = dot(h, w1h_ref[...]) + dot(aggr, w1a_ref[...]) + b1_ref[...]
        z = _ln_relu(u, g1_ref[...], be1_ref[...])
        z = _ln_relu(dot(z, w2_ref[...]) + b2_ref[...],
                     g2_ref[...], be2_ref[...])
        hn = jnp.where(deg > 0.0, h + z, h)
        hn_ref[...] = hn
        if tail:
            yn_ref[...] = dot(hn, wn_ref[...]) + bn_ref[...]

    return body


def _tc_update(h, s_parts, r_parts, ws, w_rbf, v_msg, b_rbf, b_msg,
               wd, w1h, w1a, b1, g1, be1, w2, b2, g2, be2, wn=None, bn=None):
    row = lambda i: (i, 0)
    part = lambda i: (0, i, 0)
    fixed2 = lambda i: (0, 0)
    tail = wn is not None
    in_specs = [
        pl.BlockSpec((_BLK, _D), row),
        pl.BlockSpec((2, _BLK, _D), part),
        pl.BlockSpec((2, _BLK, _RW), part),
        pl.BlockSpec((_D, _D), fixed2),
        pl.BlockSpec((_K, _D), fixed2),
        pl.BlockSpec((_D, _D), fixed2),
        pl.BlockSpec((1, _D), fixed2),
        pl.BlockSpec((1, _D), fixed2),
        pl.BlockSpec((_D, _D), fixed2),
        pl.BlockSpec((_D, _D), fixed2),
        pl.BlockSpec((_D, _D), fixed2),
        pl.BlockSpec((1, _D), fixed2),
        pl.BlockSpec((1, _D), fixed2),
        pl.BlockSpec((1, _D), fixed2),
        pl.BlockSpec((_D, _D), fixed2),
        pl.BlockSpec((1, _D), fixed2),
        pl.BlockSpec((1, _D), fixed2),
        pl.BlockSpec((1, _D), fixed2),
    ]
    args = [h, s_parts, r_parts, ws, w_rbf, v_msg, b_rbf, b_msg,
            wd, w1h, w1a, b1, g1, be1, w2, b2, g2, be2]
    out_specs = [pl.BlockSpec((_BLK, _D), row)]
    out_shape = [jax.ShapeDtypeStruct((_NP, _D), _f32)]
    if tail:
        in_specs += [pl.BlockSpec((_D, _D), fixed2),
                     pl.BlockSpec((1, _D), fixed2)]
        args += [wn, bn]
        out_specs.append(pl.BlockSpec((_BLK, _D), row))
        out_shape.append(jax.ShapeDtypeStruct((_NP, _D), _f32))
    res = pl.pallas_call(
        _make_upd_body(tail),
        grid=(_NP // _BLK,),
        in_specs=in_specs,
        out_specs=out_specs,
        out_shape=out_shape,
    )(*args)
    return res if tail else res[0]


def kernel(x, pos, edge_index, W_msg, V_msg, b_msg, W_rbf, b_rbf,
           W_u1, b_u1, g1, be1, W_u2, b_u2, g2, be2, W_out, b_out):
    src = edge_index[0].astype(_i32)
    dst = edge_index[1].astype(_i32)
    pad = _N + (jnp.arange(_EPAD - _E, dtype=_i32) % (_NP - _N))
    src2 = jnp.concatenate([src, pad]).reshape(_EPAD // _CH, _CH)
    dst2 = jnp.concatenate([dst, pad]).reshape(_EPAD // _CH, _CH)
    src2r = src2.reshape(_EPAD // _RCH, _RCH)
    dst2r = dst2.reshape(_EPAD // _RCH, _RCH)
    zpad = jnp.zeros((_NP - _N,), _f32)
    px = jnp.concatenate([pos[:, 0], zpad])
    py = jnp.concatenate([pos[:, 1], zpad])
    pz = jnp.concatenate([pos[:, 2], zpad])
    xp = jnp.zeros((_NP, _D), _f32).at[:_N].set(x)

    r_parts = _sc_rbf_deg(px, py, pz, src2r, dst2r)

    def row2(b):
        return b.reshape(1, _D)

    s_parts = _sc_scatter_rows(xp, src2, dst2)
    h1 = _tc_update(
        xp, s_parts, r_parts, W_msg[0, _D:], W_rbf[0], V_msg[0],
        row2(b_rbf[0]), row2(b_msg[0]), W_msg[0, :_D], W_u1[0, :_D],
        W_u1[0, _D:], row2(b_u1[0]), row2(g1[0]), row2(be1[0]), W_u2[0],
        row2(b_u2[0]), row2(g2[0]), row2(be2[0]))
    s_parts1 = _sc_scatter_rows(h1, src2, dst2)
    _, out = _tc_update(
        h1, s_parts1, r_parts, W_msg[1, _D:], W_rbf[1], V_msg[1],
        row2(b_rbf[1]), row2(b_msg[1]), W_msg[1, :_D], W_u1[1, :_D],
        W_u1[1, _D:], row2(b_u1[1]), row2(g1[1]), row2(be1[1]), W_u2[1],
        row2(b_u2[1]), row2(g2[1]), row2(be2[1]), W_out, row2(b_out))
    return out[:_N]
